# baseline (speedup 1.0000x reference)
import functools

import jax
import jax.numpy as jnp
from jax import lax
from jax.experimental import pallas as pl
from jax.experimental.pallas import tpu as pltpu
from jax.experimental.pallas import tpu_sc as plsc

_N = 32768
_LANES = 16
_NVEC = _N // _LANES
_UNROLL = 8
_NEG = float("-inf")

_TC_ROWS = 96
_SC_ROWS = 128 - _TC_ROWS
_TC_BLOCK = 16


def _tc_body(x_ref, v_ref, i_ref):
    x = x_ref[...]
    iota = lax.broadcasted_iota(jnp.int32, x.shape, 1)
    big = jnp.int32(2**30)
    neg = jnp.float32(-jnp.inf)
    out_iota = lax.broadcasted_iota(jnp.int32, (x.shape[0], 128), 1)
    vvals = jnp.zeros((x.shape[0], 128), jnp.float32)
    ivals = jnp.zeros((x.shape[0], 128), jnp.int32)
    for k in range(3):
        v = jnp.max(x, axis=1, keepdims=True)
        i = jnp.min(jnp.where(x == v, iota, big), axis=1, keepdims=True)
        vvals = jnp.where(out_iota == k, v, vvals)
        ivals = jnp.where(out_iota == k, i, ivals)
        if k < 2:
            x = jnp.where(iota == i, neg, x)
    v_ref[...] = vvals
    i_ref[...] = ivals


def _tc_topk(x):
    m = x.shape[0]
    grid = (m // _TC_BLOCK,)
    v, i = pl.pallas_call(
        _tc_body,
        grid=grid,
        in_specs=[pl.BlockSpec((_TC_BLOCK, _N), lambda r: (r, 0))],
        out_specs=[
            pl.BlockSpec((_TC_BLOCK, 128), lambda r: (r, 0)),
            pl.BlockSpec((_TC_BLOCK, 128), lambda r: (r, 0)),
        ],
        out_shape=[
            jax.ShapeDtypeStruct((m, 128), jnp.float32),
            jax.ShapeDtypeStruct((m, 128), jnp.int32),
        ],
    )(x)
    return v[:, :3], i[:, :3]


def _gather16(v, perm):
    return lax.gather(
        v, perm[:, None],
        dimension_numbers=lax.GatherDimensionNumbers(
            offset_dims=(), collapsed_slice_dims=(0,), start_index_map=(0,)),
        slice_sizes=(1,),
        mode=lax.GatherScatterMode.PROMISE_IN_BOUNDS)


def _sc_kernel_body(x_hbm, vout_hbm, iout_hbm, buf, outv, outi, sem0):
    nc = 2
    wid = lax.axis_index("s") * nc + lax.axis_index("c")
    iota = lax.iota(jnp.int32, _LANES)

    pltpu.async_copy(x_hbm.at[wid], buf, sem0).wait()
    row = buf

    neg = jnp.full((_LANES,), _NEG, jnp.float32)
    zero = jnp.zeros((_LANES,), jnp.int32)

    def body(c, carry):
        t1, t2, t3, i1, i2, i3, ibase = carry
        for u in range(_UNROLL):
            j = c * _UNROLL + u
            v = row[pl.ds(j * _LANES, _LANES)]
            iv = ibase + (u * _LANES)
            m1 = v > t1
            m2 = v > t2
            m3 = v > t3
            t3 = jnp.where(m2, t2, jnp.where(m3, v, t3))
            i3 = jnp.where(m2, i2, jnp.where(m3, iv, i3))
            t2 = jnp.where(m1, t1, jnp.where(m2, v, t2))
            i2 = jnp.where(m1, i1, jnp.where(m2, iv, i2))
            t1 = jnp.where(m1, v, t1)
            i1 = jnp.where(m1, iv, i1)
        ibase = ibase + (_UNROLL * _LANES)
        return t1, t2, t3, i1, i2, i3, ibase

    t1, t2, t3, i1, i2, i3, _ = lax.fori_loop(
        0, _NVEC // _UNROLL, body,
        (neg, neg, neg, zero, zero, zero, iota))

    vvec = jnp.zeros((_LANES,), jnp.float32)
    ivec = jnp.zeros((_LANES,), jnp.int32)
    for k in range(3):
        bt, bi = t1, i1
        for s in (1, 2, 4, 8):
            perm = iota ^ s
            ot = _gather16(bt, perm)
            oi = _gather16(bi, perm)
            take = (ot > bt) | ((ot == bt) & (oi < bi))
            bt = jnp.where(take, ot, bt)
            bi = jnp.where(take, oi, bi)
        vvec = jnp.where(iota == k, bt, vvec)
        ivec = jnp.where(iota == k, bi, ivec)
        if k < 2:
            hit = i1 == bi
            t1 = jnp.where(hit, t2, t1)
            i1 = jnp.where(hit, i2, i1)
            t2 = jnp.where(hit, t3, t2)
            i2 = jnp.where(hit, i3, i2)
            t3 = jnp.where(hit, jnp.float32(_NEG), t3)
    outv[0] = vvec
    outi[0] = ivec

    pltpu.sync_copy(outv, vout_hbm.at[pl.ds(wid, 1)])
    pltpu.sync_copy(outi, iout_hbm.at[pl.ds(wid, 1)])


def _sc_topk(x):
    m = x.shape[0]
    mesh = plsc.VectorSubcoreMesh(core_axis_name="c", subcore_axis_name="s")
    k = functools.partial(
        pl.kernel,
        mesh=mesh,
        out_type=[
            jax.ShapeDtypeStruct((m, _LANES), jnp.float32),
            jax.ShapeDtypeStruct((m, _LANES), jnp.int32),
        ],
        scratch_types=[
            pltpu.VMEM((_N,), jnp.float32),
            pltpu.VMEM((1, _LANES), jnp.float32),
            pltpu.VMEM((1, _LANES), jnp.int32),
            pltpu.SemaphoreType.DMA,
        ],
    )(_sc_kernel_body)
    v, i = k(x)
    return v[:, :3], i[:, :3]


def kernel(x):
    tv, ti = _tc_topk(x[:_TC_ROWS])
    sv, si = _sc_topk(x[_TC_ROWS:])
    return (jnp.concatenate([tv, sv], axis=0),
            jnp.concatenate([ti, si], axis=0))


# speedup vs baseline: 1.4022x; 1.4022x over previous
import functools

import jax
import jax.numpy as jnp
from jax import lax
from jax.experimental import pallas as pl
from jax.experimental.pallas import tpu as pltpu
from jax.experimental.pallas import tpu_sc as plsc

_N = 32768
_LANES = 16
_NVEC = _N // _LANES
_UNROLL = 8
_NEG = float("-inf")

_TC_ROWS = 96
_SC_ROWS = 128 - _TC_ROWS
_TC_BLOCK = 16
_OUTW = 8


def _tc_body(x_ref, v_ref, i_ref):
    x = x_ref[...]
    iota = lax.broadcasted_iota(jnp.int32, x.shape, 1)
    big = jnp.int32(2**30)
    neg = jnp.float32(-jnp.inf)
    out_iota = lax.broadcasted_iota(jnp.int32, (x.shape[0], _OUTW), 1)
    vvals = jnp.zeros((x.shape[0], _OUTW), jnp.float32)
    ivals = jnp.zeros((x.shape[0], _OUTW), jnp.int32)
    for k in range(3):
        v = jnp.max(x, axis=1, keepdims=True)
        i = jnp.min(jnp.where(x == v, iota, big), axis=1, keepdims=True)
        vvals = jnp.where(out_iota == k, v, vvals)
        ivals = jnp.where(out_iota == k, i, ivals)
        if k < 2:
            x = jnp.where(iota == i, neg, x)
    v_ref[...] = vvals
    i_ref[...] = ivals


def _tc_topk(x):
    grid = (_TC_ROWS // _TC_BLOCK,)
    v, i = pl.pallas_call(
        _tc_body,
        grid=grid,
        in_specs=[pl.BlockSpec((_TC_BLOCK, _N), lambda r: (r, 0))],
        out_specs=[
            pl.BlockSpec((_TC_BLOCK, _OUTW), lambda r: (r, 0)),
            pl.BlockSpec((_TC_BLOCK, _OUTW), lambda r: (r, 0)),
        ],
        out_shape=[
            jax.ShapeDtypeStruct((_TC_ROWS, _OUTW), jnp.float32),
            jax.ShapeDtypeStruct((_TC_ROWS, _OUTW), jnp.int32),
        ],
    )(x)
    return v[:, :3], i[:, :3]


def _gather16(v, perm):
    return lax.gather(
        v, perm[:, None],
        dimension_numbers=lax.GatherDimensionNumbers(
            offset_dims=(), collapsed_slice_dims=(0,), start_index_map=(0,)),
        slice_sizes=(1,),
        mode=lax.GatherScatterMode.PROMISE_IN_BOUNDS)


def _sc_kernel_body(x_hbm, vout_hbm, iout_hbm, buf, outv, outi, sem0):
    nc = 2
    wid = lax.axis_index("s") * nc + lax.axis_index("c")
    iota = lax.iota(jnp.int32, _LANES)

    pltpu.async_copy(x_hbm.at[_TC_ROWS + wid], buf, sem0).wait()
    row = buf

    neg = jnp.full((_LANES,), _NEG, jnp.float32)
    zero = jnp.zeros((_LANES,), jnp.int32)

    def body(c, carry):
        t1, t2, t3, i1, i2, i3, ibase = carry
        for u in range(_UNROLL):
            j = c * _UNROLL + u
            v = row[pl.ds(j * _LANES, _LANES)]
            iv = ibase + (u * _LANES)
            m1 = v > t1
            m2 = v > t2
            m3 = v > t3
            t3 = jnp.where(m2, t2, jnp.where(m3, v, t3))
            i3 = jnp.where(m2, i2, jnp.where(m3, iv, i3))
            t2 = jnp.where(m1, t1, jnp.where(m2, v, t2))
            i2 = jnp.where(m1, i1, jnp.where(m2, iv, i2))
            t1 = jnp.where(m1, v, t1)
            i1 = jnp.where(m1, iv, i1)
        ibase = ibase + (_UNROLL * _LANES)
        return t1, t2, t3, i1, i2, i3, ibase

    t1, t2, t3, i1, i2, i3, _ = lax.fori_loop(
        0, _NVEC // _UNROLL, body,
        (neg, neg, neg, zero, zero, zero, iota))

    vvec = jnp.zeros((_LANES,), jnp.float32)
    ivec = jnp.zeros((_LANES,), jnp.int32)
    for k in range(3):
        bt, bi = t1, i1
        for s in (1, 2, 4, 8):
            perm = iota ^ s
            ot = _gather16(bt, perm)
            oi = _gather16(bi, perm)
            take = (ot > bt) | ((ot == bt) & (oi < bi))
            bt = jnp.where(take, ot, bt)
            bi = jnp.where(take, oi, bi)
        vvec = jnp.where(iota == k, bt, vvec)
        ivec = jnp.where(iota == k, bi, ivec)
        if k < 2:
            hit = i1 == bi
            t1 = jnp.where(hit, t2, t1)
            i1 = jnp.where(hit, i2, i1)
            t2 = jnp.where(hit, t3, t2)
            i2 = jnp.where(hit, i3, i2)
            t3 = jnp.where(hit, jnp.float32(_NEG), t3)
    outv[0] = vvec
    outi[0] = ivec

    pltpu.sync_copy(outv, vout_hbm.at[pl.ds(wid, 1)])
    pltpu.sync_copy(outi, iout_hbm.at[pl.ds(wid, 1)])


def _sc_topk(x):
    mesh = plsc.VectorSubcoreMesh(core_axis_name="c", subcore_axis_name="s")
    k = functools.partial(
        pl.kernel,
        mesh=mesh,
        out_type=[
            jax.ShapeDtypeStruct((_SC_ROWS, _LANES), jnp.float32),
            jax.ShapeDtypeStruct((_SC_ROWS, _LANES), jnp.int32),
        ],
        scratch_types=[
            pltpu.VMEM((_N,), jnp.float32),
            pltpu.VMEM((1, _LANES), jnp.float32),
            pltpu.VMEM((1, _LANES), jnp.int32),
            pltpu.SemaphoreType.DMA,
        ],
    )(_sc_kernel_body)
    v, i = k(x)
    return v[:, :3], i[:, :3]


def kernel(x):
    sv, si = _sc_topk(x)
    tv, ti = _tc_topk(x)
    return (jnp.concatenate([tv, sv], axis=0),
            jnp.concatenate([ti, si], axis=0))


# speedup vs baseline: 1.4195x; 1.0123x over previous
import functools

import jax
import jax.numpy as jnp
from jax import lax
from jax.experimental import pallas as pl
from jax.experimental.pallas import tpu as pltpu
from jax.experimental.pallas import tpu_sc as plsc

_N = 32768
_LANES = 16
_NVEC = _N // _LANES
_NBLK = 64
_BVEC = _NVEC // _NBLK
_BELEM = _BVEC * _LANES
_NEG = float("-inf")

_TC_ROWS = 64
_SC_ROWS = 128 - _TC_ROWS
_ROWS_PER_TILE = _SC_ROWS // 32
_TC_BLOCK = 16
_OUTW = 8


def _tc_body(x_ref, v_ref, i_ref):
    x = x_ref[...]
    iota = lax.broadcasted_iota(jnp.int32, x.shape, 1)
    big = jnp.int32(2**30)
    neg = jnp.float32(-jnp.inf)
    out_iota = lax.broadcasted_iota(jnp.int32, (x.shape[0], _OUTW), 1)
    vvals = jnp.zeros((x.shape[0], _OUTW), jnp.float32)
    ivals = jnp.zeros((x.shape[0], _OUTW), jnp.int32)
    for k in range(3):
        v = jnp.max(x, axis=1, keepdims=True)
        i = jnp.min(jnp.where(x == v, iota, big), axis=1, keepdims=True)
        vvals = jnp.where(out_iota == k, v, vvals)
        ivals = jnp.where(out_iota == k, i, ivals)
        if k < 2:
            x = jnp.where(iota == i, neg, x)
    v_ref[...] = vvals
    i_ref[...] = ivals


def _tc_topk(x):
    grid = (_TC_ROWS // _TC_BLOCK,)
    v, i = pl.pallas_call(
        _tc_body,
        grid=grid,
        in_specs=[pl.BlockSpec((_TC_BLOCK, _N), lambda r: (r, 0))],
        out_specs=[
            pl.BlockSpec((_TC_BLOCK, _OUTW), lambda r: (r, 0)),
            pl.BlockSpec((_TC_BLOCK, _OUTW), lambda r: (r, 0)),
        ],
        out_shape=[
            jax.ShapeDtypeStruct((_TC_ROWS, _OUTW), jnp.float32),
            jax.ShapeDtypeStruct((_TC_ROWS, _OUTW), jnp.int32),
        ],
    )(x)
    return v[:, :3], i[:, :3]


def _gather16(v, perm):
    return lax.gather(
        v, perm[:, None],
        dimension_numbers=lax.GatherDimensionNumbers(
            offset_dims=(), collapsed_slice_dims=(0,), start_index_map=(0,)),
        slice_sizes=(1,),
        mode=lax.GatherScatterMode.PROMISE_IN_BOUNDS)


def _pop_best(iota, t, i, p):
    bt, bi, bp = t, i, p
    for s in (1, 2, 4, 8):
        perm = iota ^ s
        ot = _gather16(bt, perm)
        oi = _gather16(bi, perm)
        op = _gather16(bp, perm)
        take = (ot > bt) | ((ot == bt) & (oi < bi))
        bt = jnp.where(take, ot, bt)
        bi = jnp.where(take, oi, bi)
        bp = jnp.where(take, op, bp)
    return bt, bi, bp


def _sc_kernel_body(x_hbm, vout_hbm, iout_hbm,
                    buf0, buf1, sumv, sumi, outv, outi, sem0, sem1):
    nc = 2
    wid = lax.axis_index("s") * nc + lax.axis_index("c")
    base = wid * _ROWS_PER_TILE
    iota = lax.iota(jnp.int32, _LANES)
    neg = jnp.full((_LANES,), _NEG, jnp.float32)
    zero = jnp.zeros((_LANES,), jnp.int32)

    bufs = (buf0, buf1)
    copies = [
        pltpu.async_copy(x_hbm.at[_TC_ROWS + base], buf0, sem0),
        pltpu.async_copy(x_hbm.at[_TC_ROWS + base + 1], buf1, sem1),
    ]

    for r in range(_ROWS_PER_TILE):
        copies[r].wait()
        row = bufs[r]

        def p1_body(b, carry):
            t = neg
            i = zero
            iv = iota + b * _BELEM
            for j in range(_BVEC):
                v = row[pl.ds(b * _BELEM + j * _LANES, _LANES)]
                m = v > t
                t = jnp.where(m, v, t)
                i = jnp.where(m, iv, i)
                iv = iv + _LANES
            sumv[pl.ds(b * _LANES, _LANES)] = t
            sumi[pl.ds(b * _LANES, _LANES)] = i
            return carry

        lax.fori_loop(0, _NBLK, p1_body, 0)

        def s_body(c, carry):
            s1, s2, s3, r1, r2, r3, p1, p2, p3 = carry
            for u in range(4):
                j = c * 4 + u
                v = sumv[pl.ds(j * _LANES, _LANES)]
                ri = sumi[pl.ds(j * _LANES, _LANES)]
                sp = iota + j * _LANES
                m1 = v > s1
                m2 = v > s2
                m3 = v > s3
                s3 = jnp.where(m2, s2, jnp.where(m3, v, s3))
                r3 = jnp.where(m2, r2, jnp.where(m3, ri, r3))
                p3 = jnp.where(m2, p2, jnp.where(m3, sp, p3))
                s2 = jnp.where(m1, s1, jnp.where(m2, v, s2))
                r2 = jnp.where(m1, r1, jnp.where(m2, ri, r2))
                p2 = jnp.where(m1, p1, jnp.where(m2, sp, p2))
                s1 = jnp.where(m1, v, s1)
                r1 = jnp.where(m1, ri, r1)
                p1 = jnp.where(m1, sp, p1)
            return s1, s2, s3, r1, r2, r3, p1, p2, p3

        s1, s2, s3, r1, r2, r3, p1, p2, p3 = lax.fori_loop(
            0, _NBLK // 4, s_body,
            (neg, neg, neg, zero, zero, zero, zero, zero, zero))

        sel = []
        for k in range(3):
            _, br, bp = _pop_best(iota, s1, r1, p1)
            sel.append(bp)
            if k < 2:
                hit = r1 == br
                s1 = jnp.where(hit, s2, s1)
                r1 = jnp.where(hit, r2, r1)
                p1 = jnp.where(hit, p2, p1)
                s2 = jnp.where(hit, s3, s2)
                r2 = jnp.where(hit, r3, r2)
                p2 = jnp.where(hit, p3, p2)
                s3 = jnp.where(hit, neg, s3)

        blks = [lax.shift_right_logical(p, 4) for p in sel]
        b0 = jnp.minimum(jnp.minimum(blks[0], blks[1]), blks[2])
        b2 = jnp.maximum(jnp.maximum(blks[0], blks[1]), blks[2])
        b1 = (blks[0] + blks[1] + blks[2]) - b0 - b2
        t1, t2, t3 = neg, neg, neg
        i1, i2, i3 = zero, zero, zero
        for blk, g in ((b0, None), (b1, b1 != b0), (b2, b2 != b1)):
            ebase = blk[0] * _BELEM
            state0 = (t1, t2, t3, i1, i2, i3, iota + ebase)

            def rs_body(c, carry):
                t1, t2, t3, i1, i2, i3, iv = carry
                for u in range(4):
                    v = row[pl.ds(ebase + (c * 4 + u) * _LANES, _LANES)]
                    m1 = v > t1
                    m2 = v > t2
                    m3 = v > t3
                    if g is not None:
                        m1 = m1 & g
                        m2 = m2 & g
                        m3 = m3 & g
                    t3 = jnp.where(m2, t2, jnp.where(m3, v, t3))
                    i3 = jnp.where(m2, i2, jnp.where(m3, iv, i3))
                    t2 = jnp.where(m1, t1, jnp.where(m2, v, t2))
                    i2 = jnp.where(m1, i1, jnp.where(m2, iv, i2))
                    t1 = jnp.where(m1, v, t1)
                    i1 = jnp.where(m1, iv, i1)
                    iv = iv + _LANES
                return t1, t2, t3, i1, i2, i3, iv

            t1, t2, t3, i1, i2, i3, _ = lax.fori_loop(
                0, _BVEC // 4, rs_body, state0)

        vvec = jnp.zeros((_LANES,), jnp.float32)
        ivec = jnp.zeros((_LANES,), jnp.int32)
        for k in range(3):
            bt, bi, _ = _pop_best(iota, t1, i1, zero)
            vvec = jnp.where(iota == k, bt, vvec)
            ivec = jnp.where(iota == k, bi, ivec)
            if k < 2:
                hit = i1 == bi
                t1 = jnp.where(hit, t2, t1)
                i1 = jnp.where(hit, i2, i1)
                t2 = jnp.where(hit, t3, t2)
                i2 = jnp.where(hit, i3, i2)
                t3 = jnp.where(hit, neg, t3)
        outv[r] = vvec
        outi[r] = ivec

    pltpu.sync_copy(outv, vout_hbm.at[pl.ds(base, _ROWS_PER_TILE)])
    pltpu.sync_copy(outi, iout_hbm.at[pl.ds(base, _ROWS_PER_TILE)])


def _sc_topk(x):
    mesh = plsc.VectorSubcoreMesh(core_axis_name="c", subcore_axis_name="s")
    k = functools.partial(
        pl.kernel,
        mesh=mesh,
        out_type=[
            jax.ShapeDtypeStruct((_SC_ROWS, _LANES), jnp.float32),
            jax.ShapeDtypeStruct((_SC_ROWS, _LANES), jnp.int32),
        ],
        scratch_types=[
            pltpu.VMEM((_N,), jnp.float32),
            pltpu.VMEM((_N,), jnp.float32),
            pltpu.VMEM((_NBLK * _LANES,), jnp.float32),
            pltpu.VMEM((_NBLK * _LANES,), jnp.int32),
            pltpu.VMEM((_ROWS_PER_TILE, _LANES), jnp.float32),
            pltpu.VMEM((_ROWS_PER_TILE, _LANES), jnp.int32),
            pltpu.SemaphoreType.DMA,
            pltpu.SemaphoreType.DMA,
        ],
    )(_sc_kernel_body)
    v, i = k(x)
    return v[:, :3], i[:, :3]


def kernel(x):
    sv, si = _sc_topk(x)
    tv, ti = _tc_topk(x)
    return (jnp.concatenate([tv, sv], axis=0),
            jnp.concatenate([ti, si], axis=0))


# speedup vs baseline: 1.4509x; 1.0221x over previous
import functools

import jax
import jax.numpy as jnp
from jax import lax
from jax.experimental import pallas as pl
from jax.experimental.pallas import tpu as pltpu
from jax.experimental.pallas import tpu_sc as plsc

_N = 32768
_LANES = 16
_NVEC = _N // _LANES
_NBLK = 64
_BVEC = _NVEC // _NBLK
_BELEM = _BVEC * _LANES
_NEG = float("-inf")

_TC_ROWS = 64
_SC_ROWS = 128 - _TC_ROWS
_ROWS_PER_TILE = _SC_ROWS // 32
_TC_BLOCK = 32
_OUTW = 8


def _tc_body(x_ref, v_ref, i_ref):
    x = x_ref[...]
    iota = lax.broadcasted_iota(jnp.int32, x.shape, 1)
    big = jnp.int32(2**30)
    neg = jnp.float32(-jnp.inf)
    out_iota = lax.broadcasted_iota(jnp.int32, (x.shape[0], _OUTW), 1)
    vvals = jnp.zeros((x.shape[0], _OUTW), jnp.float32)
    ivals = jnp.zeros((x.shape[0], _OUTW), jnp.int32)
    for k in range(3):
        v = jnp.max(x, axis=1, keepdims=True)
        i = jnp.min(jnp.where(x == v, iota, big), axis=1, keepdims=True)
        vvals = jnp.where(out_iota == k, v, vvals)
        ivals = jnp.where(out_iota == k, i, ivals)
        if k < 2:
            x = jnp.where(iota == i, neg, x)
    v_ref[...] = vvals
    i_ref[...] = ivals


def _tc_topk(x):
    grid = (_TC_ROWS // _TC_BLOCK,)
    v, i = pl.pallas_call(
        _tc_body,
        grid=grid,
        in_specs=[pl.BlockSpec((_TC_BLOCK, _N), lambda r: (r, 0))],
        out_specs=[
            pl.BlockSpec((_TC_BLOCK, _OUTW), lambda r: (r, 0)),
            pl.BlockSpec((_TC_BLOCK, _OUTW), lambda r: (r, 0)),
        ],
        out_shape=[
            jax.ShapeDtypeStruct((_TC_ROWS, _OUTW), jnp.float32),
            jax.ShapeDtypeStruct((_TC_ROWS, _OUTW), jnp.int32),
        ],
    )(x)
    return v[:, :3], i[:, :3]


def _gather16(v, perm):
    return lax.gather(
        v, perm[:, None],
        dimension_numbers=lax.GatherDimensionNumbers(
            offset_dims=(), collapsed_slice_dims=(0,), start_index_map=(0,)),
        slice_sizes=(1,),
        mode=lax.GatherScatterMode.PROMISE_IN_BOUNDS)


def _pop_best(iota, t, i, p):
    bt, bi, bp = t, i, p
    for s in (1, 2, 4, 8):
        perm = iota ^ s
        ot = _gather16(bt, perm)
        oi = _gather16(bi, perm)
        op = _gather16(bp, perm)
        take = (ot > bt) | ((ot == bt) & (oi < bi))
        bt = jnp.where(take, ot, bt)
        bi = jnp.where(take, oi, bi)
        bp = jnp.where(take, op, bp)
    return bt, bi, bp


def _sc_kernel_body(x_hbm, vout_hbm, iout_hbm,
                    buf0, buf1, sumv, sumi, outv, outi, sem0, sem1):
    nc = 2
    wid = lax.axis_index("s") * nc + lax.axis_index("c")
    base = wid * _ROWS_PER_TILE
    iota = lax.iota(jnp.int32, _LANES)
    neg = jnp.full((_LANES,), _NEG, jnp.float32)
    zero = jnp.zeros((_LANES,), jnp.int32)

    bufs = (buf0, buf1)
    nch = 4
    chelem = _N // nch
    xrow0 = x_hbm.at[_TC_ROWS + base]
    copies = [
        pltpu.async_copy(xrow0.at[pl.ds(c * chelem, chelem)],
                         buf0.at[pl.ds(c * chelem, chelem)], sem0)
        for c in range(nch)
    ]
    copy1 = pltpu.async_copy(x_hbm.at[_TC_ROWS + base + 1], buf1, sem1)

    for r in range(_ROWS_PER_TILE):
        row = bufs[r]

        def p1_body(b, carry):
            t = neg
            i = zero
            iv = iota + b * _BELEM
            for j in range(_BVEC):
                v = row[pl.ds(b * _BELEM + j * _LANES, _LANES)]
                m = v > t
                t = jnp.where(m, v, t)
                i = jnp.where(m, iv, i)
                iv = iv + _LANES
            sumv[pl.ds(b * _LANES, _LANES)] = t
            sumi[pl.ds(b * _LANES, _LANES)] = i
            return carry

        if r == 0:
            blk_per_ch = _NBLK // nch
            for c in range(nch):
                copies[c].wait()
                lax.fori_loop(c * blk_per_ch, (c + 1) * blk_per_ch,
                              p1_body, 0)
        else:
            copy1.wait()
            lax.fori_loop(0, _NBLK, p1_body, 0)

        def s_body(c, carry):
            s1, s2, s3, r1, r2, r3, p1, p2, p3 = carry
            for u in range(4):
                j = c * 4 + u
                v = sumv[pl.ds(j * _LANES, _LANES)]
                ri = sumi[pl.ds(j * _LANES, _LANES)]
                sp = iota + j * _LANES
                m1 = v > s1
                m2 = v > s2
                m3 = v > s3
                s3 = jnp.where(m2, s2, jnp.where(m3, v, s3))
                r3 = jnp.where(m2, r2, jnp.where(m3, ri, r3))
                p3 = jnp.where(m2, p2, jnp.where(m3, sp, p3))
                s2 = jnp.where(m1, s1, jnp.where(m2, v, s2))
                r2 = jnp.where(m1, r1, jnp.where(m2, ri, r2))
                p2 = jnp.where(m1, p1, jnp.where(m2, sp, p2))
                s1 = jnp.where(m1, v, s1)
                r1 = jnp.where(m1, ri, r1)
                p1 = jnp.where(m1, sp, p1)
            return s1, s2, s3, r1, r2, r3, p1, p2, p3

        s1, s2, s3, r1, r2, r3, p1, p2, p3 = lax.fori_loop(
            0, _NBLK // 4, s_body,
            (neg, neg, neg, zero, zero, zero, zero, zero, zero))

        sel = []
        for k in range(3):
            _, br, bp = _pop_best(iota, s1, r1, p1)
            sel.append(bp)
            if k < 2:
                hit = r1 == br
                s1 = jnp.where(hit, s2, s1)
                r1 = jnp.where(hit, r2, r1)
                p1 = jnp.where(hit, p2, p1)
                s2 = jnp.where(hit, s3, s2)
                r2 = jnp.where(hit, r3, r2)
                p2 = jnp.where(hit, p3, p2)
                s3 = jnp.where(hit, neg, s3)

        blks = [lax.shift_right_logical(p, 4) for p in sel]
        b0 = jnp.minimum(jnp.minimum(blks[0], blks[1]), blks[2])
        b2 = jnp.maximum(jnp.maximum(blks[0], blks[1]), blks[2])
        b1 = (blks[0] + blks[1] + blks[2]) - b0 - b2
        t1, t2, t3 = neg, neg, neg
        i1, i2, i3 = zero, zero, zero
        for blk, g in ((b0, None), (b1, b1 != b0), (b2, b2 != b1)):
            ebase = blk[0] * _BELEM
            state0 = (t1, t2, t3, i1, i2, i3, iota + ebase)

            def rs_body(c, carry):
                t1, t2, t3, i1, i2, i3, iv = carry
                for u in range(4):
                    v = row[pl.ds(ebase + (c * 4 + u) * _LANES, _LANES)]
                    m1 = v > t1
                    m2 = v > t2
                    m3 = v > t3
                    if g is not None:
                        m1 = m1 & g
                        m2 = m2 & g
                        m3 = m3 & g
                    t3 = jnp.where(m2, t2, jnp.where(m3, v, t3))
                    i3 = jnp.where(m2, i2, jnp.where(m3, iv, i3))
                    t2 = jnp.where(m1, t1, jnp.where(m2, v, t2))
                    i2 = jnp.where(m1, i1, jnp.where(m2, iv, i2))
                    t1 = jnp.where(m1, v, t1)
                    i1 = jnp.where(m1, iv, i1)
                    iv = iv + _LANES
                return t1, t2, t3, i1, i2, i3, iv

            t1, t2, t3, i1, i2, i3, _ = lax.fori_loop(
                0, _BVEC // 4, rs_body, state0)

        vvec = jnp.zeros((_LANES,), jnp.float32)
        ivec = jnp.zeros((_LANES,), jnp.int32)
        for k in range(3):
            bt, bi, _ = _pop_best(iota, t1, i1, zero)
            vvec = jnp.where(iota == k, bt, vvec)
            ivec = jnp.where(iota == k, bi, ivec)
            if k < 2:
                hit = i1 == bi
                t1 = jnp.where(hit, t2, t1)
                i1 = jnp.where(hit, i2, i1)
                t2 = jnp.where(hit, t3, t2)
                i2 = jnp.where(hit, i3, i2)
                t3 = jnp.where(hit, neg, t3)
        outv[r] = vvec
        outi[r] = ivec

    pltpu.sync_copy(outv, vout_hbm.at[pl.ds(base, _ROWS_PER_TILE)])
    pltpu.sync_copy(outi, iout_hbm.at[pl.ds(base, _ROWS_PER_TILE)])


def _sc_topk(x):
    mesh = plsc.VectorSubcoreMesh(core_axis_name="c", subcore_axis_name="s")
    k = functools.partial(
        pl.kernel,
        mesh=mesh,
        out_type=[
            jax.ShapeDtypeStruct((_SC_ROWS, _LANES), jnp.float32),
            jax.ShapeDtypeStruct((_SC_ROWS, _LANES), jnp.int32),
        ],
        scratch_types=[
            pltpu.VMEM((_N,), jnp.float32),
            pltpu.VMEM((_N,), jnp.float32),
            pltpu.VMEM((_NBLK * _LANES,), jnp.float32),
            pltpu.VMEM((_NBLK * _LANES,), jnp.int32),
            pltpu.VMEM((_ROWS_PER_TILE, _LANES), jnp.float32),
            pltpu.VMEM((_ROWS_PER_TILE, _LANES), jnp.int32),
            pltpu.SemaphoreType.DMA,
            pltpu.SemaphoreType.DMA,
        ],
    )(_sc_kernel_body)
    v, i = k(x)
    return v[:, :3], i[:, :3]


def kernel(x):
    sv, si = _sc_topk(x)
    tv, ti = _tc_topk(x)
    return (jnp.concatenate([tv, sv], axis=0),
            jnp.concatenate([ti, si], axis=0))


# speedup vs baseline: 1.4584x; 1.0052x over previous
import functools

import jax
import jax.numpy as jnp
from jax import lax
from jax.experimental import pallas as pl
from jax.experimental.pallas import tpu as pltpu
from jax.experimental.pallas import tpu_sc as plsc

_N = 32768
_LANES = 16
_NVEC = _N // _LANES
_NBLK = 64
_BVEC = _NVEC // _NBLK
_BELEM = _BVEC * _LANES
_NEG = float("-inf")

_TC_ROWS = 64
_SC_ROWS = 128 - _TC_ROWS
_ROWS_PER_TILE = _SC_ROWS // 32
_TC_BLOCK = 32
_OUTW = 8


def _tc_body(x_ref, v_ref, i_ref):
    x = x_ref[...]
    iota = lax.broadcasted_iota(jnp.int32, x.shape, 1)
    big = jnp.int32(2**30)
    neg = jnp.float32(-jnp.inf)
    out_iota = lax.broadcasted_iota(jnp.int32, (x.shape[0], _OUTW), 1)
    vvals = jnp.zeros((x.shape[0], _OUTW), jnp.float32)
    ivals = jnp.zeros((x.shape[0], _OUTW), jnp.int32)
    for k in range(3):
        v = jnp.max(x, axis=1, keepdims=True)
        i = jnp.min(jnp.where(x == v, iota, big), axis=1, keepdims=True)
        vvals = jnp.where(out_iota == k, v, vvals)
        ivals = jnp.where(out_iota == k, i, ivals)
        if k < 2:
            x = jnp.where(iota == i, neg, x)
    v_ref[...] = vvals
    i_ref[...] = ivals


def _tc_topk(x):
    grid = (_TC_ROWS // _TC_BLOCK,)
    v, i = pl.pallas_call(
        _tc_body,
        grid=grid,
        in_specs=[pl.BlockSpec((_TC_BLOCK, _N), lambda r: (r, 0))],
        out_specs=[
            pl.BlockSpec((_TC_BLOCK, _OUTW), lambda r: (r, 0)),
            pl.BlockSpec((_TC_BLOCK, _OUTW), lambda r: (r, 0)),
        ],
        out_shape=[
            jax.ShapeDtypeStruct((_TC_ROWS, _OUTW), jnp.float32),
            jax.ShapeDtypeStruct((_TC_ROWS, _OUTW), jnp.int32),
        ],
    )(x)
    return v[:, :3], i[:, :3]


def _gather16(v, perm):
    return lax.gather(
        v, perm[:, None],
        dimension_numbers=lax.GatherDimensionNumbers(
            offset_dims=(), collapsed_slice_dims=(0,), start_index_map=(0,)),
        slice_sizes=(1,),
        mode=lax.GatherScatterMode.PROMISE_IN_BOUNDS)


def _pop_best(iota, t, i, p):
    bt, bi, bp = t, i, p
    for s in (1, 2, 4, 8):
        perm = iota ^ s
        ot = _gather16(bt, perm)
        oi = _gather16(bi, perm)
        op = _gather16(bp, perm)
        take = (ot > bt) | ((ot == bt) & (oi < bi))
        bt = jnp.where(take, ot, bt)
        bi = jnp.where(take, oi, bi)
        bp = jnp.where(take, op, bp)
    return bt, bi, bp


def _sc_kernel_body(x_hbm, vout_hbm, iout_hbm,
                    buf0, buf1, sumv, sumi, outv, outi, sem0, sem1):
    nc = 2
    wid = lax.axis_index("s") * nc + lax.axis_index("c")
    base = wid * _ROWS_PER_TILE
    iota = lax.iota(jnp.int32, _LANES)
    neg = jnp.full((_LANES,), _NEG, jnp.float32)
    zero = jnp.zeros((_LANES,), jnp.int32)

    bufs = (buf0, buf1)
    nch = 4
    chelem = _N // nch
    xrow0 = x_hbm.at[_TC_ROWS + base]
    copies = [
        pltpu.async_copy(xrow0.at[pl.ds(c * chelem, chelem)],
                         buf0.at[pl.ds(c * chelem, chelem)], sem0)
        for c in range(nch)
    ]
    copy1 = pltpu.async_copy(x_hbm.at[_TC_ROWS + base + 1], buf1, sem1)

    for r in range(_ROWS_PER_TILE):
        row = bufs[r]

        def p1_body(b, carry):
            t = neg
            i = zero
            iv = iota + b * _BELEM
            for j in range(_BVEC):
                v = row[pl.ds(b * _BELEM + j * _LANES, _LANES)]
                m = v > t
                t = jnp.where(m, v, t)
                i = jnp.where(m, iv, i)
                iv = iv + _LANES
            sumv[pl.ds(b * _LANES, _LANES)] = t
            sumi[pl.ds(b * _LANES, _LANES)] = i
            return carry

        if r == 0:
            blk_per_ch = _NBLK // nch
            for c in range(nch):
                copies[c].wait()
                lax.fori_loop(c * blk_per_ch, (c + 1) * blk_per_ch,
                              p1_body, 0)
        else:
            copy1.wait()
            lax.fori_loop(0, _NBLK, p1_body, 0)

        def s_body(c, carry):
            s1, s2, s3, r1, r2, r3, p1, p2, p3 = carry
            for u in range(4):
                j = c * 4 + u
                v = sumv[pl.ds(j * _LANES, _LANES)]
                ri = sumi[pl.ds(j * _LANES, _LANES)]
                sp = iota + j * _LANES
                m1 = v > s1
                m2 = v > s2
                m3 = v > s3
                s3 = jnp.where(m2, s2, jnp.where(m3, v, s3))
                r3 = jnp.where(m2, r2, jnp.where(m3, ri, r3))
                p3 = jnp.where(m2, p2, jnp.where(m3, sp, p3))
                s2 = jnp.where(m1, s1, jnp.where(m2, v, s2))
                r2 = jnp.where(m1, r1, jnp.where(m2, ri, r2))
                p2 = jnp.where(m1, p1, jnp.where(m2, sp, p2))
                s1 = jnp.where(m1, v, s1)
                r1 = jnp.where(m1, ri, r1)
                p1 = jnp.where(m1, sp, p1)
            return s1, s2, s3, r1, r2, r3, p1, p2, p3

        s1, s2, s3, r1, r2, r3, p1, p2, p3 = lax.fori_loop(
            0, _NBLK // 4, s_body,
            (neg, neg, neg, zero, zero, zero, zero, zero, zero))

        sel = []
        for k in range(3):
            _, br, bp = _pop_best(iota, s1, r1, p1)
            sel.append(bp)
            if k < 2:
                hit = r1 == br
                s1 = jnp.where(hit, s2, s1)
                r1 = jnp.where(hit, r2, r1)
                p1 = jnp.where(hit, p2, p1)
                s2 = jnp.where(hit, s3, s2)
                r2 = jnp.where(hit, r3, r2)
                p2 = jnp.where(hit, p3, p2)
                s3 = jnp.where(hit, neg, s3)

        blks = [lax.shift_right_logical(p, 4) for p in sel]
        b0 = jnp.minimum(jnp.minimum(blks[0], blks[1]), blks[2])
        b2 = jnp.maximum(jnp.maximum(blks[0], blks[1]), blks[2])
        b1 = (blks[0] + blks[1] + blks[2]) - b0 - b2
        t1, t2, t3 = neg, neg, neg
        i1, i2, i3 = zero, zero, zero
        for blk, g in ((b0, None), (b1, b1 != b0), (b2, b2 != b1)):
            ebase = blk[0] * _BELEM
            state0 = (t1, t2, t3, i1, i2, i3, iota + ebase)

            def rs_body(c, carry):
                t1, t2, t3, i1, i2, i3, iv = carry
                for u in range(4):
                    v = row[pl.ds(ebase + (c * 4 + u) * _LANES, _LANES)]
                    m1 = v > t1
                    m2 = v > t2
                    m3 = v > t3
                    if g is not None:
                        m1 = m1 & g
                        m2 = m2 & g
                        m3 = m3 & g
                    t3 = jnp.where(m2, t2, jnp.where(m3, v, t3))
                    i3 = jnp.where(m2, i2, jnp.where(m3, iv, i3))
                    t2 = jnp.where(m1, t1, jnp.where(m2, v, t2))
                    i2 = jnp.where(m1, i1, jnp.where(m2, iv, i2))
                    t1 = jnp.where(m1, v, t1)
                    i1 = jnp.where(m1, iv, i1)
                    iv = iv + _LANES
                return t1, t2, t3, i1, i2, i3, iv

            t1, t2, t3, i1, i2, i3, _ = lax.fori_loop(
                0, _BVEC // 4, rs_body, state0)

        vvec = jnp.zeros((_LANES,), jnp.float32)
        ivec = jnp.zeros((_LANES,), jnp.int32)
        for k in range(3):
            bt, bi, _ = _pop_best(iota, t1, i1, zero)
            vvec = jnp.where(iota == k, bt, vvec)
            ivec = jnp.where(iota == k, bi, ivec)
            if k < 2:
                hit = i1 == bi
                t1 = jnp.where(hit, t2, t1)
                i1 = jnp.where(hit, i2, i1)
                t2 = jnp.where(hit, t3, t2)
                i2 = jnp.where(hit, i3, i2)
                t3 = jnp.where(hit, neg, t3)
        outv[r] = vvec
        outi[r] = ivec

    pltpu.sync_copy(outv, vout_hbm.at[pl.ds(base, _ROWS_PER_TILE)])
    pltpu.sync_copy(outi, iout_hbm.at[pl.ds(base, _ROWS_PER_TILE)])


def _sc_topk(x):
    mesh = plsc.VectorSubcoreMesh(core_axis_name="c", subcore_axis_name="s")
    k = functools.partial(
        pl.kernel,
        mesh=mesh,
        out_type=[
            jax.ShapeDtypeStruct((_SC_ROWS, _LANES), jnp.float32),
            jax.ShapeDtypeStruct((_SC_ROWS, _LANES), jnp.int32),
        ],
        scratch_types=[
            pltpu.VMEM((_N,), jnp.float32),
            pltpu.VMEM((_N,), jnp.float32),
            pltpu.VMEM((_NBLK * _LANES,), jnp.float32),
            pltpu.VMEM((_NBLK * _LANES,), jnp.int32),
            pltpu.VMEM((_ROWS_PER_TILE, _LANES), jnp.float32),
            pltpu.VMEM((_ROWS_PER_TILE, _LANES), jnp.int32),
            pltpu.SemaphoreType.DMA,
            pltpu.SemaphoreType.DMA,
        ],
    )(_sc_kernel_body)
    v, i = k(x)
    return v[:, :3], i[:, :3]


def kernel(x):
    tv, ti = _tc_topk(x)
    sv, si = _sc_topk(x)
    return (jnp.concatenate([tv, sv], axis=0),
            jnp.concatenate([ti, si], axis=0))


# speedup vs baseline: 1.4584x; 1.0000x over previous
import functools

import jax
import jax.numpy as jnp
from jax import lax
from jax.experimental import pallas as pl
from jax.experimental.pallas import tpu as pltpu
from jax.experimental.pallas import tpu_sc as plsc

_N = 32768
_LANES = 16
_NVEC = _N // _LANES
_NBLK = 64
_BVEC = _NVEC // _NBLK
_BELEM = _BVEC * _LANES
_NEG = float("-inf")

_TC_ROWS = 64
_SC_ROWS = 128 - _TC_ROWS
_ROWS_PER_TILE = _SC_ROWS // 32
_TC_BLOCK = 32
_OUTW = 8


def _tc_body(x_ref, v_ref, i_ref):
    x = x_ref[...]
    iota = lax.broadcasted_iota(jnp.int32, x.shape, 1)
    big = jnp.int32(2**30)
    neg = jnp.float32(-jnp.inf)
    out_iota = lax.broadcasted_iota(jnp.int32, (x.shape[0], _OUTW), 1)
    vvals = jnp.zeros((x.shape[0], _OUTW), jnp.float32)
    ivals = jnp.zeros((x.shape[0], _OUTW), jnp.int32)
    for k in range(3):
        v = jnp.max(x, axis=1, keepdims=True)
        i = jnp.min(jnp.where(x == v, iota, big), axis=1, keepdims=True)
        vvals = jnp.where(out_iota == k, v, vvals)
        ivals = jnp.where(out_iota == k, i, ivals)
        if k < 2:
            x = jnp.where(iota == i, neg, x)
    v_ref[...] = vvals
    i_ref[...] = ivals


def _tc_topk(x):
    grid = (_TC_ROWS // _TC_BLOCK,)
    v, i = pl.pallas_call(
        _tc_body,
        grid=grid,
        in_specs=[pl.BlockSpec((_TC_BLOCK, _N), lambda r: (r, 0))],
        out_specs=[
            pl.BlockSpec((_TC_BLOCK, _OUTW), lambda r: (r, 0)),
            pl.BlockSpec((_TC_BLOCK, _OUTW), lambda r: (r, 0)),
        ],
        out_shape=[
            jax.ShapeDtypeStruct((_TC_ROWS, _OUTW), jnp.float32),
            jax.ShapeDtypeStruct((_TC_ROWS, _OUTW), jnp.int32),
        ],
    )(x)
    return v[:, :3], i[:, :3]


def _gather16(v, perm):
    return lax.gather(
        v, perm[:, None],
        dimension_numbers=lax.GatherDimensionNumbers(
            offset_dims=(), collapsed_slice_dims=(0,), start_index_map=(0,)),
        slice_sizes=(1,),
        mode=lax.GatherScatterMode.PROMISE_IN_BOUNDS)


def _pop_best(iota, t, i, p=None):
    bt, bi, bp = t, i, p
    for s in (1, 2, 4, 8):
        perm = iota ^ s
        ot = _gather16(bt, perm)
        oi = _gather16(bi, perm)
        take = (ot > bt) | ((ot == bt) & (oi < bi))
        if bp is not None:
            bp = jnp.where(take, _gather16(bp, perm), bp)
        bt = jnp.where(take, ot, bt)
        bi = jnp.where(take, oi, bi)
    return bt, bi, bp


def _sc_kernel_body(x_hbm, vout_hbm, iout_hbm,
                    buf0, buf1, sumv, sumi, outv, outi, sem0, sem1):
    nc = 2
    wid = lax.axis_index("s") * nc + lax.axis_index("c")
    base = wid * _ROWS_PER_TILE
    iota = lax.iota(jnp.int32, _LANES)
    neg = jnp.full((_LANES,), _NEG, jnp.float32)
    zero = jnp.zeros((_LANES,), jnp.int32)

    bufs = (buf0, buf1)
    nch = 4
    chelem = _N // nch
    xrow0 = x_hbm.at[_TC_ROWS + base]
    copies = [
        pltpu.async_copy(xrow0.at[pl.ds(c * chelem, chelem)],
                         buf0.at[pl.ds(c * chelem, chelem)], sem0)
        for c in range(nch)
    ]
    copy1 = pltpu.async_copy(x_hbm.at[_TC_ROWS + base + 1], buf1, sem1)

    for r in range(_ROWS_PER_TILE):
        row = bufs[r]

        def p1_body(b, carry):
            t = neg
            i = zero
            iv = iota + b * _BELEM
            for j in range(_BVEC):
                v = row[pl.ds(b * _BELEM + j * _LANES, _LANES)]
                m = v > t
                t = jnp.where(m, v, t)
                i = jnp.where(m, iv, i)
                iv = iv + _LANES
            sumv[pl.ds(b * _LANES, _LANES)] = t
            sumi[pl.ds(b * _LANES, _LANES)] = i
            return carry

        if r == 0:
            blk_per_ch = _NBLK // nch
            for c in range(nch):
                copies[c].wait()
                lax.fori_loop(c * blk_per_ch, (c + 1) * blk_per_ch,
                              p1_body, 0)
        else:
            copy1.wait()
            lax.fori_loop(0, _NBLK, p1_body, 0)

        def s_body(c, carry):
            s1, s2, s3, r1, r2, r3, p1, p2, p3 = carry
            for u in range(4):
                j = c * 4 + u
                v = sumv[pl.ds(j * _LANES, _LANES)]
                ri = sumi[pl.ds(j * _LANES, _LANES)]
                sp = iota + j * _LANES
                m1 = v > s1
                m2 = v > s2
                m3 = v > s3
                s3 = jnp.where(m2, s2, jnp.where(m3, v, s3))
                r3 = jnp.where(m2, r2, jnp.where(m3, ri, r3))
                p3 = jnp.where(m2, p2, jnp.where(m3, sp, p3))
                s2 = jnp.where(m1, s1, jnp.where(m2, v, s2))
                r2 = jnp.where(m1, r1, jnp.where(m2, ri, r2))
                p2 = jnp.where(m1, p1, jnp.where(m2, sp, p2))
                s1 = jnp.where(m1, v, s1)
                r1 = jnp.where(m1, ri, r1)
                p1 = jnp.where(m1, sp, p1)
            return s1, s2, s3, r1, r2, r3, p1, p2, p3

        s1, s2, s3, r1, r2, r3, p1, p2, p3 = lax.fori_loop(
            0, _NBLK // 4, s_body,
            (neg, neg, neg, zero, zero, zero, zero, zero, zero))

        sel = []
        for k in range(3):
            _, br, bp = _pop_best(iota, s1, r1, p1)
            sel.append(bp)
            if k < 2:
                hit = r1 == br
                s1 = jnp.where(hit, s2, s1)
                r1 = jnp.where(hit, r2, r1)
                p1 = jnp.where(hit, p2, p1)
                s2 = jnp.where(hit, s3, s2)
                r2 = jnp.where(hit, r3, r2)
                p2 = jnp.where(hit, p3, p2)
                s3 = jnp.where(hit, neg, s3)

        blks = [lax.shift_right_logical(p, 4) for p in sel]
        b0 = jnp.minimum(jnp.minimum(blks[0], blks[1]), blks[2])
        b2 = jnp.maximum(jnp.maximum(blks[0], blks[1]), blks[2])
        b1 = (blks[0] + blks[1] + blks[2]) - b0 - b2
        t1, t2, t3 = neg, neg, neg
        i1, i2, i3 = zero, zero, zero
        for blk, g in ((b0, None), (b1, b1 != b0), (b2, b2 != b1)):
            ebase = blk[0] * _BELEM
            state0 = (t1, t2, t3, i1, i2, i3, iota + ebase)

            def rs_body(c, carry):
                t1, t2, t3, i1, i2, i3, iv = carry
                for u in range(4):
                    v = row[pl.ds(ebase + (c * 4 + u) * _LANES, _LANES)]
                    m1 = v > t1
                    m2 = v > t2
                    m3 = v > t3
                    if g is not None:
                        m1 = m1 & g
                        m2 = m2 & g
                        m3 = m3 & g
                    t3 = jnp.where(m2, t2, jnp.where(m3, v, t3))
                    i3 = jnp.where(m2, i2, jnp.where(m3, iv, i3))
                    t2 = jnp.where(m1, t1, jnp.where(m2, v, t2))
                    i2 = jnp.where(m1, i1, jnp.where(m2, iv, i2))
                    t1 = jnp.where(m1, v, t1)
                    i1 = jnp.where(m1, iv, i1)
                    iv = iv + _LANES
                return t1, t2, t3, i1, i2, i3, iv

            t1, t2, t3, i1, i2, i3, _ = lax.fori_loop(
                0, _BVEC // 4, rs_body, state0)

        vvec = jnp.zeros((_LANES,), jnp.float32)
        ivec = jnp.zeros((_LANES,), jnp.int32)
        for k in range(3):
            bt, bi, _ = _pop_best(iota, t1, i1)
            vvec = jnp.where(iota == k, bt, vvec)
            ivec = jnp.where(iota == k, bi, ivec)
            if k < 2:
                hit = i1 == bi
                t1 = jnp.where(hit, t2, t1)
                i1 = jnp.where(hit, i2, i1)
                t2 = jnp.where(hit, t3, t2)
                i2 = jnp.where(hit, i3, i2)
                t3 = jnp.where(hit, neg, t3)
        outv[r] = vvec
        outi[r] = ivec

    pltpu.sync_copy(outv, vout_hbm.at[pl.ds(base, _ROWS_PER_TILE)])
    pltpu.sync_copy(outi, iout_hbm.at[pl.ds(base, _ROWS_PER_TILE)])


def _sc_topk(x):
    mesh = plsc.VectorSubcoreMesh(core_axis_name="c", subcore_axis_name="s")
    k = functools.partial(
        pl.kernel,
        mesh=mesh,
        out_type=[
            jax.ShapeDtypeStruct((_SC_ROWS, _LANES), jnp.float32),
            jax.ShapeDtypeStruct((_SC_ROWS, _LANES), jnp.int32),
        ],
        scratch_types=[
            pltpu.VMEM((_N,), jnp.float32),
            pltpu.VMEM((_N,), jnp.float32),
            pltpu.VMEM((_NBLK * _LANES,), jnp.float32),
            pltpu.VMEM((_NBLK * _LANES,), jnp.int32),
            pltpu.VMEM((_ROWS_PER_TILE, _LANES), jnp.float32),
            pltpu.VMEM((_ROWS_PER_TILE, _LANES), jnp.int32),
            pltpu.SemaphoreType.DMA,
            pltpu.SemaphoreType.DMA,
        ],
    )(_sc_kernel_body)
    v, i = k(x)
    return v[:, :3], i[:, :3]


def kernel(x):
    tv, ti = _tc_topk(x)
    sv, si = _sc_topk(x)
    return (jnp.concatenate([tv, sv], axis=0),
            jnp.concatenate([ti, si], axis=0))


# speedup vs baseline: 1.4801x; 1.0148x over previous
import functools

import jax
import jax.numpy as jnp
from jax import lax
from jax.experimental import pallas as pl
from jax.experimental.pallas import tpu as pltpu
from jax.experimental.pallas import tpu_sc as plsc

_N = 32768
_LANES = 16
_NVEC = _N // _LANES
_NBLK = 64
_BVEC = _NVEC // _NBLK
_BELEM = _BVEC * _LANES
_NEG = float("-inf")

_TC_ROWS = 64
_SC_ROWS = 128 - _TC_ROWS
_ROWS_PER_TILE = _SC_ROWS // 32
_TC_BLOCK = 32
_OUTW = 8


def _tc_body(x_ref, v_ref, i_ref):
    x = x_ref[...]
    iota = lax.broadcasted_iota(jnp.int32, x.shape, 1)
    big = jnp.int32(2**30)
    neg = jnp.float32(-jnp.inf)
    out_iota = lax.broadcasted_iota(jnp.int32, (x.shape[0], _OUTW), 1)
    vvals = jnp.zeros((x.shape[0], _OUTW), jnp.float32)
    ivals = jnp.zeros((x.shape[0], _OUTW), jnp.int32)
    for k in range(3):
        v = jnp.max(x, axis=1, keepdims=True)
        i = jnp.min(jnp.where(x == v, iota, big), axis=1, keepdims=True)
        vvals = jnp.where(out_iota == k, v, vvals)
        ivals = jnp.where(out_iota == k, i, ivals)
        if k < 2:
            x = jnp.where(iota == i, neg, x)
    v_ref[...] = vvals
    i_ref[...] = ivals


def _tc_topk(x):
    grid = (_TC_ROWS // _TC_BLOCK,)
    v, i = pl.pallas_call(
        _tc_body,
        grid=grid,
        in_specs=[pl.BlockSpec((_TC_BLOCK, _N), lambda r: (r, 0))],
        out_specs=[
            pl.BlockSpec((_TC_BLOCK, _OUTW), lambda r: (r, 0)),
            pl.BlockSpec((_TC_BLOCK, _OUTW), lambda r: (r, 0)),
        ],
        out_shape=[
            jax.ShapeDtypeStruct((_TC_ROWS, _OUTW), jnp.float32),
            jax.ShapeDtypeStruct((_TC_ROWS, _OUTW), jnp.int32),
        ],
    )(x)
    return v[:, :3], i[:, :3]


def _gather16(v, perm):
    return lax.gather(
        v, perm[:, None],
        dimension_numbers=lax.GatherDimensionNumbers(
            offset_dims=(), collapsed_slice_dims=(0,), start_index_map=(0,)),
        slice_sizes=(1,),
        mode=lax.GatherScatterMode.PROMISE_IN_BOUNDS)


def _pop_best(iota, t, i, p=None):
    bt, bi, bp = t, i, p
    for s in (1, 2, 4, 8):
        perm = iota ^ s
        ot = _gather16(bt, perm)
        oi = _gather16(bi, perm)
        take = (ot > bt) | ((ot == bt) & (oi < bi))
        if bp is not None:
            bp = jnp.where(take, _gather16(bp, perm), bp)
        bt = jnp.where(take, ot, bt)
        bi = jnp.where(take, oi, bi)
    return bt, bi, bp


def _sc_kernel_body(x_hbm, vout_hbm, iout_hbm,
                    buf0, buf1, outv, outi, sem0, sem1):
    nc = 2
    wid = lax.axis_index("s") * nc + lax.axis_index("c")
    base = wid * _ROWS_PER_TILE
    iota = lax.iota(jnp.int32, _LANES)
    neg = jnp.full((_LANES,), _NEG, jnp.float32)
    zero = jnp.zeros((_LANES,), jnp.int32)

    bufs = (buf0, buf1)
    nch = 4
    chelem = _N // nch
    xrow0 = x_hbm.at[_TC_ROWS + base]
    copies = [
        pltpu.async_copy(xrow0.at[pl.ds(c * chelem, chelem)],
                         buf0.at[pl.ds(c * chelem, chelem)], sem0)
        for c in range(nch)
    ]
    copy1 = pltpu.async_copy(x_hbm.at[_TC_ROWS + base + 1], buf1, sem1)

    for r in range(_ROWS_PER_TILE):
        row = bufs[r]

        def p1_body(b, carry):
            s1, s2, s3, r1, r2, r3, p1, p2, p3 = carry
            t = neg
            i = zero
            iv = iota + b * _BELEM
            for j in range(_BVEC):
                v = row[pl.ds(b * _BELEM + j * _LANES, _LANES)]
                m = v > t
                t = jnp.where(m, v, t)
                i = jnp.where(m, iv, i)
                iv = iv + _LANES
            sp = iota + b * _LANES
            m1 = t > s1
            m2 = t > s2
            m3 = t > s3
            s3 = jnp.where(m2, s2, jnp.where(m3, t, s3))
            r3 = jnp.where(m2, r2, jnp.where(m3, i, r3))
            p3 = jnp.where(m2, p2, jnp.where(m3, sp, p3))
            s2 = jnp.where(m1, s1, jnp.where(m2, t, s2))
            r2 = jnp.where(m1, r1, jnp.where(m2, i, r2))
            p2 = jnp.where(m1, p1, jnp.where(m2, sp, p2))
            s1 = jnp.where(m1, t, s1)
            r1 = jnp.where(m1, i, r1)
            p1 = jnp.where(m1, sp, p1)
            return s1, s2, s3, r1, r2, r3, p1, p2, p3

        carry0 = (neg, neg, neg, zero, zero, zero, zero, zero, zero)
        if r == 0:
            blk_per_ch = _NBLK // nch
            for c in range(nch):
                copies[c].wait()
                carry0 = lax.fori_loop(c * blk_per_ch, (c + 1) * blk_per_ch,
                                       p1_body, carry0)
        else:
            copy1.wait()
            carry0 = lax.fori_loop(0, _NBLK, p1_body, carry0)
        s1, s2, s3, r1, r2, r3, p1, p2, p3 = carry0

        sel = []
        for k in range(3):
            _, br, bp = _pop_best(iota, s1, r1, p1)
            sel.append(bp)
            if k < 2:
                hit = r1 == br
                s1 = jnp.where(hit, s2, s1)
                r1 = jnp.where(hit, r2, r1)
                p1 = jnp.where(hit, p2, p1)
                s2 = jnp.where(hit, s3, s2)
                r2 = jnp.where(hit, r3, r2)
                p2 = jnp.where(hit, p3, p2)
                s3 = jnp.where(hit, neg, s3)

        blks = [lax.shift_right_logical(p, 4) for p in sel]
        b0 = jnp.minimum(jnp.minimum(blks[0], blks[1]), blks[2])
        b2 = jnp.maximum(jnp.maximum(blks[0], blks[1]), blks[2])
        b1 = (blks[0] + blks[1] + blks[2]) - b0 - b2
        t1, t2, t3 = neg, neg, neg
        i1, i2, i3 = zero, zero, zero
        for blk, g in ((b0, None), (b1, b1 != b0), (b2, b2 != b1)):
            ebase = blk[0] * _BELEM
            state0 = (t1, t2, t3, i1, i2, i3, iota + ebase)

            def rs_body(c, carry):
                t1, t2, t3, i1, i2, i3, iv = carry
                for u in range(4):
                    v = row[pl.ds(ebase + (c * 4 + u) * _LANES, _LANES)]
                    m1 = v > t1
                    m2 = v > t2
                    m3 = v > t3
                    if g is not None:
                        m1 = m1 & g
                        m2 = m2 & g
                        m3 = m3 & g
                    t3 = jnp.where(m2, t2, jnp.where(m3, v, t3))
                    i3 = jnp.where(m2, i2, jnp.where(m3, iv, i3))
                    t2 = jnp.where(m1, t1, jnp.where(m2, v, t2))
                    i2 = jnp.where(m1, i1, jnp.where(m2, iv, i2))
                    t1 = jnp.where(m1, v, t1)
                    i1 = jnp.where(m1, iv, i1)
                    iv = iv + _LANES
                return t1, t2, t3, i1, i2, i3, iv

            t1, t2, t3, i1, i2, i3, _ = lax.fori_loop(
                0, _BVEC // 4, rs_body, state0)

        vvec = jnp.zeros((_LANES,), jnp.float32)
        ivec = jnp.zeros((_LANES,), jnp.int32)
        for k in range(3):
            bt, bi, _ = _pop_best(iota, t1, i1)
            vvec = jnp.where(iota == k, bt, vvec)
            ivec = jnp.where(iota == k, bi, ivec)
            if k < 2:
                hit = i1 == bi
                t1 = jnp.where(hit, t2, t1)
                i1 = jnp.where(hit, i2, i1)
                t2 = jnp.where(hit, t3, t2)
                i2 = jnp.where(hit, i3, i2)
                t3 = jnp.where(hit, neg, t3)
        outv[r] = vvec
        outi[r] = ivec

    pltpu.sync_copy(outv, vout_hbm.at[pl.ds(base, _ROWS_PER_TILE)])
    pltpu.sync_copy(outi, iout_hbm.at[pl.ds(base, _ROWS_PER_TILE)])


def _sc_topk(x):
    mesh = plsc.VectorSubcoreMesh(core_axis_name="c", subcore_axis_name="s")
    k = functools.partial(
        pl.kernel,
        mesh=mesh,
        out_type=[
            jax.ShapeDtypeStruct((_SC_ROWS, _LANES), jnp.float32),
            jax.ShapeDtypeStruct((_SC_ROWS, _LANES), jnp.int32),
        ],
        scratch_types=[
            pltpu.VMEM((_N,), jnp.float32),
            pltpu.VMEM((_N,), jnp.float32),
            pltpu.VMEM((_ROWS_PER_TILE, _LANES), jnp.float32),
            pltpu.VMEM((_ROWS_PER_TILE, _LANES), jnp.int32),
            pltpu.SemaphoreType.DMA,
            pltpu.SemaphoreType.DMA,
        ],
    )(_sc_kernel_body)
    v, i = k(x)
    return v[:, :3], i[:, :3]


def kernel(x):
    tv, ti = _tc_topk(x)
    sv, si = _sc_topk(x)
    return (jnp.concatenate([tv, sv], axis=0),
            jnp.concatenate([ti, si], axis=0))
